# baseline (speedup 1.0000x reference)
import functools

import jax
import jax.numpy as jnp
from jax import lax
from jax.experimental import pallas as pl
from jax.experimental.pallas import tpu as pltpu
from jax.experimental.pallas import tpu_sc as plsc

_INFO = plsc.get_sparse_core_info()
_NC = _INFO.num_cores
_NS = _INFO.num_subcores
_NW = _NC * _NS

_BCHUNK = 8
_NBUF = 4
_LEAD = 2


def _make_gather(b: int, s: int, d: int):
    per_w = b // _NW
    n_chunks = per_w // _BCHUNK
    rows = _BCHUNK * s
    assert per_w % _BCHUNK == 0 and n_chunks % _NBUF == 0 and n_chunks >= 2 * _NBUF

    mesh = plsc.VectorSubcoreMesh(core_axis_name="c", subcore_axis_name="s")

    @functools.partial(
        pl.kernel,
        out_type=jax.ShapeDtypeStruct((b, s, d), jnp.float32),
        mesh=mesh,
        scratch_types=(
            [pltpu.VMEM((n_chunks, rows), jnp.int32)]
            + [pltpu.VMEM((rows, d), jnp.float32) for _ in range(_NBUF)]
            + [pltpu.SemaphoreType.DMA for _ in range(2 * _NBUF)]
        ),
        compiler_params=pltpu.CompilerParams(use_tc_tiling_on_sc=False),
    )
    def gather_kernel(table_hbm, idx_hbm, out_hbm, idx_v, *bufs_and_sems):
        bufs = bufs_and_sems[:_NBUF]
        sg = bufs_and_sems[_NBUF:2 * _NBUF]
        so = bufs_and_sems[2 * _NBUF:3 * _NBUF]

        wid = lax.axis_index("s") * _NC + lax.axis_index("c")
        pltpu.sync_copy(idx_hbm.at[wid], idx_v)
        base = wid * per_w

        def fire_gather(k, slot):
            pltpu.async_copy(table_hbm.at[idx_v.at[k]], bufs[slot], sg[slot])

        def wait_gather(k, slot):
            pltpu.make_async_copy(
                table_hbm.at[idx_v.at[k]], bufs[slot], sg[slot]).wait()

        def fire_copy(c, slot):
            for i in range(_BCHUNK):
                pltpu.async_copy(
                    bufs[slot].at[pl.ds(i * s, s)],
                    out_hbm.at[base + c * _BCHUNK + i], so[slot])

        def wait_copy(c, slot):
            for i in range(_BCHUNK):
                pltpu.make_async_copy(
                    bufs[slot].at[pl.ds(i * s, s)],
                    out_hbm.at[base + c * _BCHUNK + i], so[slot]).wait()

        def body(k, b_slot, wait_prev_copy, fire_next):
            wait_gather(k, b_slot)
            fire_copy(k, b_slot)
            if fire_next:
                bn = (b_slot + _LEAD) % _NBUF
                if wait_prev_copy:
                    wait_copy(k + _LEAD - _NBUF, bn)
                fire_gather(k + _LEAD, bn)

        for c in range(_LEAD):
            fire_gather(c, c)
        for i in range(_NBUF):
            body(i, i, wait_prev_copy=(i >= _NBUF - _LEAD), fire_next=True)

        @pl.loop(_NBUF, n_chunks - _NBUF, step=_NBUF)
        def _(g):
            for i in range(_NBUF):
                body(g + i, i, wait_prev_copy=True, fire_next=True)

        for i in range(_NBUF):
            k = n_chunks - _NBUF + i
            body(k, i, wait_prev_copy=True, fire_next=(i < _NBUF - _LEAD))
        for i in range(_NBUF):
            wait_copy(n_chunks - _NBUF + i, i)

    return gather_kernel


def kernel(input, table):
    b, s = input.shape
    v, d = table.shape
    per_w = b // _NW
    n_chunks = per_w // _BCHUNK
    idx3d = input.reshape(_NW, n_chunks, _BCHUNK * s).astype(jnp.int32)
    return _make_gather(b, s, d)(table, idx3d)


# speedup vs baseline: 1.0061x; 1.0061x over previous
import functools

import jax
import jax.numpy as jnp
from jax import lax
from jax.experimental import pallas as pl
from jax.experimental.pallas import tpu as pltpu
from jax.experimental.pallas import tpu_sc as plsc

_INFO = plsc.get_sparse_core_info()
_NC = _INFO.num_cores
_NS = _INFO.num_subcores
_NW = _NC * _NS

_BCHUNK = 4
_NBUF = 8
_LEAD = 4


def _make_gather(b: int, s: int, d: int):
    per_w = b // _NW
    n_chunks = per_w // _BCHUNK
    rows = _BCHUNK * s
    assert per_w % _BCHUNK == 0 and n_chunks % _NBUF == 0 and n_chunks >= 2 * _NBUF

    mesh = plsc.VectorSubcoreMesh(core_axis_name="c", subcore_axis_name="s")

    @functools.partial(
        pl.kernel,
        out_type=jax.ShapeDtypeStruct((b, s, d), jnp.float32),
        mesh=mesh,
        scratch_types=(
            [pltpu.VMEM((n_chunks, rows), jnp.int32)]
            + [pltpu.VMEM((rows, d), jnp.float32) for _ in range(_NBUF)]
            + [pltpu.SemaphoreType.DMA for _ in range(2 * _NBUF)]
        ),
        compiler_params=pltpu.CompilerParams(use_tc_tiling_on_sc=False),
    )
    def gather_kernel(table_hbm, idx_hbm, out_hbm, idx_v, *bufs_and_sems):
        bufs = bufs_and_sems[:_NBUF]
        sg = bufs_and_sems[_NBUF:2 * _NBUF]
        so = bufs_and_sems[2 * _NBUF:3 * _NBUF]

        wid = lax.axis_index("s") * _NC + lax.axis_index("c")
        pltpu.sync_copy(idx_hbm.at[wid], idx_v)
        base = wid * per_w

        def fire_gather(k, slot):
            pltpu.async_copy(table_hbm.at[idx_v.at[k]], bufs[slot], sg[slot])

        def wait_gather(k, slot):
            pltpu.make_async_copy(
                table_hbm.at[idx_v.at[k]], bufs[slot], sg[slot]).wait()

        def fire_copy(c, slot):
            for i in range(_BCHUNK):
                pltpu.async_copy(
                    bufs[slot].at[pl.ds(i * s, s)],
                    out_hbm.at[base + c * _BCHUNK + i], so[slot])

        def wait_copy(c, slot):
            for i in range(_BCHUNK):
                pltpu.make_async_copy(
                    bufs[slot].at[pl.ds(i * s, s)],
                    out_hbm.at[base + c * _BCHUNK + i], so[slot]).wait()

        def body(k, b_slot, wait_prev_copy, fire_next):
            wait_gather(k, b_slot)
            fire_copy(k, b_slot)
            if fire_next:
                bn = (b_slot + _LEAD) % _NBUF
                if wait_prev_copy:
                    wait_copy(k + _LEAD - _NBUF, bn)
                fire_gather(k + _LEAD, bn)

        for c in range(_LEAD):
            fire_gather(c, c)
        for i in range(_NBUF):
            body(i, i, wait_prev_copy=(i >= _NBUF - _LEAD), fire_next=True)

        @pl.loop(_NBUF, n_chunks - _NBUF, step=_NBUF)
        def _(g):
            for i in range(_NBUF):
                body(g + i, i, wait_prev_copy=True, fire_next=True)

        for i in range(_NBUF):
            k = n_chunks - _NBUF + i
            body(k, i, wait_prev_copy=True, fire_next=(i < _NBUF - _LEAD))
        for i in range(_NBUF):
            wait_copy(n_chunks - _NBUF + i, i)

    return gather_kernel


def kernel(input, table):
    b, s = input.shape
    v, d = table.shape
    per_w = b // _NW
    n_chunks = per_w // _BCHUNK
    idx3d = input.reshape(_NW, n_chunks, _BCHUNK * s).astype(jnp.int32)
    return _make_gather(b, s, d)(table, idx3d)
